# Initial kernel scaffold; baseline (speedup 1.0000x reference)
import jax
import jax.numpy as jnp
from jax.experimental import pallas as pl


def kernel(x, edge_index, W1, b1, W2, b2):
    raise NotImplementedError("write your pallas kernel here")


# speedup vs baseline: 11.0077x; 11.0077x over previous
import functools

import jax
import jax.numpy as jnp
from jax import lax
from jax.experimental import pallas as pl
from jax.experimental.pallas import tpu as pltpu
from jax.experimental.pallas import tpu_sc as plsc

N_NODES = 10000
D = 128
E = 320000

NC = 2
NS = 16
NW = NC * NS

B = 128
NPAD = 10240
PAD_IDX = N_NODES
EPAD = ((E + NW * B - 1) // (NW * B)) * (NW * B)
EW = EPAD // NW
NB = EW // B
NPT = NPAD // NS

RB = 1024
GRID = NPAD // RB

_mesh = plsc.VectorSubcoreMesh(core_axis_name="c", subcore_axis_name="s")


@functools.partial(
    pl.kernel,
    out_type=(
        jax.ShapeDtypeStruct((NPAD,), jnp.float32),
        jax.ShapeDtypeStruct((NPAD,), jnp.float32),
    ),
    mesh=_mesh,
    scratch_types=(
        pltpu.VMEM_SHARED((NPAD,), jnp.float32),
        pltpu.VMEM((B,), jnp.int32),
        pltpu.VMEM((B,), jnp.float32),
        pltpu.SemaphoreType.DMA,
    ),
)
def _deg_kernel(col_hbm, ones_hbm, zeros_hbm, out0, out1, acc_sh, cidx_v,
                ones_v, sem):
    c = lax.axis_index("c")
    s = lax.axis_index("s")
    w = s * NC + c
    base = w * EW
    lo = s * NPT

    @pl.when(c == 0)
    def _():
        pltpu.sync_copy(ones_hbm.at[pl.ds(lo, NPT)], acc_sh.at[pl.ds(lo, NPT)])

    @pl.when(c != 0)
    def _():
        pltpu.sync_copy(zeros_hbm.at[pl.ds(lo, NPT)], acc_sh.at[pl.ds(lo, NPT)])

    pltpu.sync_copy(ones_hbm.at[pl.ds(0, B)], ones_v)
    plsc.subcore_barrier()

    def body(j, carry):
        pltpu.sync_copy(col_hbm.at[pl.ds(base + j * B, B)], cidx_v)
        pltpu.sync_copy(ones_v, acc_sh.at[cidx_v], add=True)
        return carry

    lax.fori_loop(0, NB, body, 0)
    plsc.subcore_barrier()

    @pl.when(c == 0)
    def _():
        pltpu.sync_copy(acc_sh.at[pl.ds(lo, NPT)], out0.at[pl.ds(lo, NPT)])

    @pl.when(c != 0)
    def _():
        pltpu.sync_copy(acc_sh.at[pl.ds(lo, NPT)], out1.at[pl.ds(lo, NPT)])


@functools.partial(
    pl.kernel,
    out_type=(
        jax.ShapeDtypeStruct((NPAD, D), jnp.float32),
        jax.ShapeDtypeStruct((NPAD, D), jnp.float32),
    ),
    mesh=_mesh,
    scratch_types=(
        pltpu.VMEM_SHARED((NPAD, D), jnp.float32),
        pltpu.VMEM((B,), jnp.int32),
        pltpu.VMEM((B,), jnp.int32),
        pltpu.VMEM((B, D), jnp.float32),
        pltpu.SemaphoreType.DMA,
    ),
)
def _agg_kernel(h_hbm, row_hbm, col_hbm, zeros_hbm, out0, out1, acc_sh,
                ridx_v, cidx_v, rows_v, sem):
    c = lax.axis_index("c")
    s = lax.axis_index("s")
    w = s * NC + c
    base = w * EW
    lo = s * NPT

    @pl.when(c == 0)
    def _():
        pltpu.sync_copy(h_hbm.at[pl.ds(lo, NPT)], acc_sh.at[pl.ds(lo, NPT)])

    @pl.when(c != 0)
    def _():
        pltpu.sync_copy(zeros_hbm.at[pl.ds(lo, NPT)], acc_sh.at[pl.ds(lo, NPT)])

    plsc.subcore_barrier()

    def body(j, carry):
        off = base + j * B
        pltpu.sync_copy(row_hbm.at[pl.ds(off, B)], ridx_v)
        pltpu.async_copy(h_hbm.at[ridx_v], rows_v, sem).wait()
        pltpu.sync_copy(col_hbm.at[pl.ds(off, B)], cidx_v)
        pltpu.sync_copy(rows_v, acc_sh.at[cidx_v], add=True)
        return carry

    lax.fori_loop(0, NB, body, 0)
    plsc.subcore_barrier()

    @pl.when(c == 0)
    def _():
        pltpu.sync_copy(acc_sh.at[pl.ds(lo, NPT)], out0.at[pl.ds(lo, NPT)])

    @pl.when(c != 0)
    def _():
        pltpu.sync_copy(acc_sh.at[pl.ds(lo, NPT)], out1.at[pl.ds(lo, NPT)])


def _mm_body(x_ref, w_ref, o_ref):
    o_ref[...] = jnp.dot(x_ref[...], w_ref[...],
                         preferred_element_type=jnp.float32)


_mm_call = pl.pallas_call(
    _mm_body,
    grid=(GRID,),
    in_specs=[
        pl.BlockSpec((RB, D), lambda i: (i, 0)),
        pl.BlockSpec((D, D), lambda i: (0, 0)),
    ],
    out_specs=pl.BlockSpec((RB, D), lambda i: (i, 0)),
    out_shape=jax.ShapeDtypeStruct((NPAD, D), jnp.float32),
)


def _scale_body(d0_ref, d1_ref, xw_ref, dinv_ref, h_ref):
    deg = d0_ref[...] + d1_ref[...]
    dinv = lax.rsqrt(deg)
    dinv_ref[...] = dinv
    h_ref[...] = xw_ref[...] * dinv[:, None]


_scale_call = pl.pallas_call(
    _scale_body,
    grid=(GRID,),
    in_specs=[
        pl.BlockSpec((RB,), lambda i: (i,)),
        pl.BlockSpec((RB,), lambda i: (i,)),
        pl.BlockSpec((RB, D), lambda i: (i, 0)),
    ],
    out_specs=[
        pl.BlockSpec((RB,), lambda i: (i,)),
        pl.BlockSpec((RB, D), lambda i: (i, 0)),
    ],
    out_shape=[
        jax.ShapeDtypeStruct((NPAD,), jnp.float32),
        jax.ShapeDtypeStruct((NPAD, D), jnp.float32),
    ],
)


def _fused_body(a0_ref, a1_ref, dinv_ref, b_ref, w_ref, o_ref):
    dinv = dinv_ref[...]
    t = (a0_ref[...] + a1_ref[...]) * dinv[:, None] + b_ref[...]
    t = jnp.maximum(t, 0.0)
    z = jnp.dot(t, w_ref[...], preferred_element_type=jnp.float32)
    o_ref[...] = z * dinv[:, None]


_fused_call = pl.pallas_call(
    _fused_body,
    grid=(GRID,),
    in_specs=[
        pl.BlockSpec((RB, D), lambda i: (i, 0)),
        pl.BlockSpec((RB, D), lambda i: (i, 0)),
        pl.BlockSpec((RB,), lambda i: (i,)),
        pl.BlockSpec((1, D), lambda i: (0, 0)),
        pl.BlockSpec((D, D), lambda i: (0, 0)),
    ],
    out_specs=pl.BlockSpec((RB, D), lambda i: (i, 0)),
    out_shape=jax.ShapeDtypeStruct((NPAD, D), jnp.float32),
)


def _final_body(a0_ref, a1_ref, dinv_ref, b_ref, o_ref):
    o_ref[...] = ((a0_ref[...] + a1_ref[...]) * dinv_ref[...][:, None]
                  + b_ref[...])


_final_call = pl.pallas_call(
    _final_body,
    grid=(GRID,),
    in_specs=[
        pl.BlockSpec((RB, D), lambda i: (i, 0)),
        pl.BlockSpec((RB, D), lambda i: (i, 0)),
        pl.BlockSpec((RB,), lambda i: (i,)),
        pl.BlockSpec((1, D), lambda i: (0, 0)),
    ],
    out_specs=pl.BlockSpec((RB, D), lambda i: (i, 0)),
    out_shape=jax.ShapeDtypeStruct((NPAD, D), jnp.float32),
)


def kernel(x, edge_index, W1, b1, W2, b2):
    row = edge_index[0].astype(jnp.int32)
    col = edge_index[1].astype(jnp.int32)
    pad = EPAD - E
    rowp = jnp.concatenate([row, jnp.full((pad,), PAD_IDX, jnp.int32)])
    colp = jnp.concatenate([col, jnp.full((pad,), PAD_IDX, jnp.int32)])
    xp = jnp.pad(x, ((0, NPAD - N_NODES), (0, 0)))
    zeros2d = jnp.zeros((NPAD, D), jnp.float32)
    ones1d = jnp.ones((NPAD,), jnp.float32)
    zeros1d = jnp.zeros((NPAD,), jnp.float32)
    b1r = b1.reshape(1, D)
    b2r = b2.reshape(1, D)

    deg0, deg1 = _deg_kernel(colp, ones1d, zeros1d)
    xw1 = _mm_call(xp, W1)
    dinv, h1s = _scale_call(deg0, deg1, xw1)

    a0, a1 = _agg_kernel(h1s, rowp, colp, zeros2d)
    h2s = _fused_call(a0, a1, dinv, b1r, W2)
    c0, c1 = _agg_kernel(h2s, rowp, colp, zeros2d)
    out = _final_call(c0, c1, dinv, b2r)
    return out[:N_NODES]
